# baseline (speedup 1.0000x reference)
import functools

import jax
import jax.numpy as jnp
from jax import lax
from jax.experimental import pallas as pl
from jax.experimental.pallas import tpu as pltpu
from jax.experimental.pallas import tpu_sc as plsc

B, T, D_IN, D_OUT, R, E = 32, 1, 4096, 4096, 16, 16
TILE_O = 512
NT = D_OUT // TILE_O
NBUF = 4
NC = 2
NL = 16

_sc_mesh = plsc.VectorSubcoreMesh(core_axis_name="c", subcore_axis_name="s")


def _tc_mid_body(x_ref, ids_ref, a_ref, out_ref):
    mid_all = jax.lax.dot_general(
        x_ref[...], a_ref[...], (((1,), (1,)), ((), ())),
        preferred_element_type=jnp.float32,
    )
    col_e = jax.lax.broadcasted_iota(jnp.int32, (B, E * R), 1) // R
    mid_m = jnp.where(col_e == ids_ref[...], mid_all, 0.0)
    krow = jax.lax.broadcasted_iota(jnp.int32, (E * R, R * NL), 0) % R
    rcol = jax.lax.broadcasted_iota(jnp.int32, (E * R, R * NL), 1) // NL
    sel = jnp.where(krow == rcol, 1.0, 0.0).astype(jnp.float32)
    out_ref[...] = jax.lax.dot_general(
        mid_m, sel, (((1,), (0,)), ((), ())),
        preferred_element_type=jnp.float32,
    )


@functools.partial(
    pl.kernel,
    out_type=jax.ShapeDtypeStruct((B, D_OUT), jnp.float32),
    mesh=_sc_mesh,
    scratch_types=[
        pltpu.VMEM((R,), jnp.int32),
        pltpu.VMEM((R * NL,), jnp.float32),
        pltpu.VMEM((R, D_OUT), jnp.float32),
        pltpu.VMEM((D_OUT,), jnp.float32),
        pltpu.SemaphoreType.DMA,
    ],
)
def _sc_delta(midrep_hbm, idx_hbm, bt_hbm, delta_hbm,
              idxv, mrepv, rowsv, outv, sem):
    wid = lax.axis_index("s") * NC + lax.axis_index("c")
    pltpu.sync_copy(idx_hbm.at[wid], idxv)
    pltpu.sync_copy(midrep_hbm.at[wid], mrepv)

    pltpu.async_copy(bt_hbm.at[idxv], rowsv, sem).wait()
    msplat = [mrepv[pl.ds(r * NL, NL)] for r in range(R)]

    def delta_body(c, carry):
        acc = msplat[0] * rowsv[0, pl.ds(c * NL, NL)]
        for r in range(1, R):
            acc = acc + msplat[r] * rowsv[r, pl.ds(c * NL, NL)]
        outv[pl.ds(c * NL, NL)] = acc
        return carry

    lax.fori_loop(0, D_OUT // NL, delta_body, 0)
    pltpu.sync_copy(outv, delta_hbm.at[wid])


def _tc_h_body(x_ref, w_hbm, out_ref, w_buf, w_sems):
    def w_copy(j, slot):
        return pltpu.make_async_copy(
            w_hbm.at[pl.ds(j * TILE_O, TILE_O), :],
            w_buf.at[slot],
            w_sems.at[slot],
        )

    for s in range(NBUF):
        w_copy(s, s).start()
    xb = x_ref[...].astype(jnp.bfloat16)
    for j in range(NT):
        slot = j % NBUF
        w_copy(j, slot).wait()
        h = jax.lax.dot_general(
            xb, w_buf[slot].astype(jnp.bfloat16), (((1,), (1,)), ((), ())),
            preferred_element_type=jnp.float32,
        )
        nxt = j + NBUF
        if nxt < NT:
            w_copy(nxt, slot).start()
        out_ref[:, pl.ds(j * TILE_O, TILE_O)] = h


def _combine_body(h_ref, d_ref, o_ref):
    o_ref[...] = h_ref[...] + d_ref[...]


@jax.jit
def kernel(x, adapter_ids, W_base, A_all, B_all):
    x2 = x.reshape(B, D_IN)
    a2 = A_all.reshape(E * R, D_IN)
    bt = jnp.swapaxes(B_all, 1, 2).reshape(E * R, D_OUT)
    ids2 = adapter_ids.reshape(B, 1).astype(jnp.int32)
    row_idx = (adapter_ids.astype(jnp.int32)[:, None] * R
               + jnp.arange(R, dtype=jnp.int32)[None, :])

    midrep = pl.pallas_call(
        _tc_mid_body,
        in_specs=[
            pl.BlockSpec((B, D_IN), lambda: (0, 0)),
            pl.BlockSpec((B, 1), lambda: (0, 0)),
            pl.BlockSpec((E * R, D_IN), lambda: (0, 0)),
        ],
        out_specs=pl.BlockSpec((B, R * NL), lambda: (0, 0)),
        out_shape=jax.ShapeDtypeStruct((B, R * NL), jnp.float32),
    )(x2, ids2, a2)

    delta = _sc_delta(midrep, row_idx, bt)

    h = pl.pallas_call(
        _tc_h_body,
        in_specs=[
            pl.BlockSpec((B, D_IN), lambda: (0, 0)),
            pl.BlockSpec(memory_space=pltpu.MemorySpace.HBM),
        ],
        out_specs=pl.BlockSpec((B, D_OUT), lambda: (0, 0)),
        out_shape=jax.ShapeDtypeStruct((B, D_OUT), jnp.float32),
        scratch_shapes=[
            pltpu.VMEM((NBUF, TILE_O, D_IN), jnp.float32),
            pltpu.SemaphoreType.DMA((NBUF,)),
        ],
    )(x2, W_base)

    out = pl.pallas_call(
        _combine_body,
        in_specs=[
            pl.BlockSpec((B, D_OUT), lambda: (0, 0)),
            pl.BlockSpec((B, D_OUT), lambda: (0, 0)),
        ],
        out_specs=pl.BlockSpec((B, D_OUT), lambda: (0, 0)),
        out_shape=jax.ShapeDtypeStruct((B, D_OUT), jnp.float32),
    )(h, delta)
    return out.reshape(B, T, D_OUT)


# speedup vs baseline: 1.6908x; 1.6908x over previous
import jax
import jax.numpy as jnp
from jax.experimental import pallas as pl
from jax.experimental.pallas import tpu as pltpu

B, T, D_IN, D_OUT, R, E = 32, 1, 4096, 4096, 16, 16
TILE_O = 512
NT = D_OUT // TILE_O
NBUF = 4


def _body(x_ref, ids_ref, a_hbm, w_hbm, b_hbm, out_ref,
          w_buf, a_vmem, b_vmem, mid_ref, w_sems, a_sem, b_sem):
    def w_copy(j, slot):
        return pltpu.make_async_copy(
            w_hbm.at[pl.ds(j * TILE_O, TILE_O), :],
            w_buf.at[slot],
            w_sems.at[slot],
        )

    a_copy = pltpu.make_async_copy(a_hbm, a_vmem, a_sem)
    b_copy = pltpu.make_async_copy(b_hbm, b_vmem, b_sem)
    a_copy.start()
    b_copy.start()
    for s in range(NBUF):
        w_copy(s, s).start()

    a_copy.wait()
    xb = x_ref[...].astype(jnp.bfloat16)
    mid_all = jax.lax.dot_general(
        x_ref[...], a_vmem[...], (((1,), (1,)), ((), ())),
        preferred_element_type=jnp.float32,
    )
    col_e = jax.lax.broadcasted_iota(jnp.int32, (B, E * R), 1) // R
    mid_ref[...] = jnp.where(col_e == ids_ref[...], mid_all, 0.0)

    b_copy.wait()
    out_ref[...] = jax.lax.dot_general(
        mid_ref[...], b_vmem[...], (((1,), (0,)), ((), ())),
        preferred_element_type=jnp.float32,
    )

    for j in range(NT):
        slot = j % NBUF
        w_copy(j, slot).wait()
        h = jax.lax.dot_general(
            xb, w_buf[slot].astype(jnp.bfloat16), (((1,), (1,)), ((), ())),
            preferred_element_type=jnp.float32,
        )
        nxt = j + NBUF
        if nxt < NT:
            w_copy(nxt, slot).start()
        out_ref[:, pl.ds(j * TILE_O, TILE_O)] += h


@jax.jit
def kernel(x, adapter_ids, W_base, A_all, B_all):
    x2 = x.reshape(B, D_IN)
    a2 = A_all.reshape(E * R, D_IN)
    b_r = jnp.swapaxes(B_all, 1, 2).reshape(E * R, D_OUT)
    ids2 = adapter_ids.reshape(B, 1).astype(jnp.int32)
    out = pl.pallas_call(
        _body,
        in_specs=[
            pl.BlockSpec((B, D_IN), lambda: (0, 0)),
            pl.BlockSpec((B, 1), lambda: (0, 0)),
            pl.BlockSpec(memory_space=pltpu.MemorySpace.HBM),
            pl.BlockSpec(memory_space=pltpu.MemorySpace.HBM),
            pl.BlockSpec(memory_space=pltpu.MemorySpace.HBM),
        ],
        out_specs=pl.BlockSpec((B, D_OUT), lambda: (0, 0)),
        out_shape=jax.ShapeDtypeStruct((B, D_OUT), jnp.float32),
        scratch_shapes=[
            pltpu.VMEM((NBUF, TILE_O, D_IN), jnp.float32),
            pltpu.VMEM((E * R, D_IN), jnp.float32),
            pltpu.VMEM((E * R, D_OUT), jnp.float32),
            pltpu.VMEM((B, E * R), jnp.float32),
            pltpu.SemaphoreType.DMA((NBUF,)),
            pltpu.SemaphoreType.DMA,
            pltpu.SemaphoreType.DMA,
        ],
    )(x2, ids2, a2, W_base, b_r)
    return out.reshape(B, T, D_OUT)
